# baseline (speedup 1.0000x reference)
import functools

import jax
import jax.numpy as jnp
from jax import lax
from jax.experimental import pallas as pl
from jax.experimental.pallas import tpu as pltpu
from jax.experimental.pallas import tpu_sc as plsc

VOCAB = 1000000
EMB = 64
B = 16384
L = 50
HID = 128

NC = 2
NS = 16
NW = NC * NS
BPW = B // NW
GCH = 128
NK = BPW // GCH


def _sc_bag_sums(x_flat, emb_table):
    mesh = plsc.VectorSubcoreMesh(core_axis_name="c", subcore_axis_name="s")

    @functools.partial(
        pl.kernel,
        out_type=jax.ShapeDtypeStruct((B, EMB), jnp.float32),
        mesh=mesh,
        name="bag_sums",
        compiler_params=pltpu.CompilerParams(
            use_tc_tiling_on_sc=False, needs_layout_passes=False
        ),
        scratch_types=[
            pltpu.VMEM((BPW * L,), jnp.int32),
            pltpu.VMEM((L, NK, GCH), jnp.int32),
            pltpu.VMEM((BPW, EMB), jnp.float32),
            pltpu.SemaphoreType.DMA,
        ],
    )
    def kern(x_hbm, tab_hbm, out_hbm, raw_v, idx_v, acc_v, sem):
        wid = lax.axis_index("s") * NC + lax.axis_index("c")
        pltpu.sync_copy(x_hbm.at[pl.ds(wid * (BPW * L), BPW * L)], raw_v)

        lane = lax.iota(jnp.int32, 16) * L

        def transpose_row(j):
            for kk in range(NK):
                for g in range(GCH // 16):
                    v = plsc.load_gather(
                        raw_v, [lane + (kk * GCH + g * 16) * L + j]
                    )
                    pv = (
                        (v & jnp.int32(~1023))
                        + ((v & jnp.int32(511)) << 1)
                        + ((v >> 9) & jnp.int32(1))
                    )
                    idx_v[j, kk, pl.ds(g * 16, 16)] = pv

        transpose_row(0)
        cps = [
            pltpu.async_copy(
                tab_hbm.at[idx_v.at[0, kk]],
                acc_v.at[pl.ds(kk * GCH, GCH)],
                sem,
            )
            for kk in range(NK)
        ]
        for cp in cps:
            cp.wait()

        @pl.loop(1, L)
        def _(j):
            transpose_row(j)
            for kk in range(NK):
                pltpu.async_copy(
                    tab_hbm.at[idx_v.at[j, kk]],
                    acc_v.at[pl.ds(kk * GCH, GCH)],
                    sem,
                    add=True,
                )

        @pl.loop(1, L)
        def _(j):
            for kk in range(NK):
                pltpu.make_async_copy(
                    tab_hbm.at[idx_v.at[0, kk]],
                    acc_v.at[pl.ds(kk * GCH, GCH)],
                    sem,
                ).wait()

        pltpu.sync_copy(acc_v, out_hbm.at[pl.ds(wid * BPW, BPW)])

    return kern(x_flat, emb_table)


VBLK = 1024
NGRID = (VOCAB + VBLK - 1) // VBLK
VPAD = NGRID * VBLK


def _tc_detile(tabT):
    def body(t_ref, o_ref):
        t = t_ref[...].T
        y = jnp.concatenate([t[: VBLK // 2, :], t[VBLK // 2 :, :]], axis=1)
        o_ref[...] = y.reshape(VBLK * EMB)

    return pl.pallas_call(
        body,
        grid=(NGRID,),
        in_specs=[pl.BlockSpec((EMB, VBLK), lambda i: (0, i))],
        out_specs=pl.BlockSpec((VBLK * EMB,), lambda i: (i,)),
        out_shape=jax.ShapeDtypeStruct((VPAD * EMB,), jnp.float32),
    )(tabT)


def _tc_mlp(sums, W1, b1, W2, b2):
    BLK = 2048

    def body(s_ref, w1_ref, b1_ref, w2_ref, b2_ref, o_ref):
        e = s_ref[...] * (1.0 / L)
        h = jnp.dot(e, w1_ref[...], preferred_element_type=jnp.float32)
        h = jnp.maximum(h + b1_ref[...], 0.0)
        p = jnp.dot(h, w2_ref[...], preferred_element_type=jnp.float32)
        o_ref[...] = jax.nn.sigmoid(p + b2_ref[...])

    return pl.pallas_call(
        body,
        grid=(B // BLK,),
        in_specs=[
            pl.BlockSpec((BLK, EMB), lambda i: (i, 0)),
            pl.BlockSpec((EMB, HID), lambda i: (0, 0)),
            pl.BlockSpec((1, HID), lambda i: (0, 0)),
            pl.BlockSpec((HID, 1), lambda i: (0, 0)),
            pl.BlockSpec((1, 1), lambda i: (0, 0)),
        ],
        out_specs=pl.BlockSpec((BLK, 1), lambda i: (i, 0)),
        out_shape=jax.ShapeDtypeStruct((B, 1), jnp.float32),
    )(sums, W1, b1, W2, b2)


def kernel(x, emb_table, W1, b1, W2, b2):
    tab_rm = _tc_detile(emb_table.T).reshape(VPAD, EMB)
    sums = _sc_bag_sums(x.reshape(B * L), tab_rm)
    return _tc_mlp(sums, W1, b1.reshape(1, HID), W2, b2.reshape(1, 1))


# speedup vs baseline: 1.7992x; 1.7992x over previous
import functools

import jax
import jax.numpy as jnp
from jax import lax
from jax.experimental import pallas as pl
from jax.experimental.pallas import tpu as pltpu
from jax.experimental.pallas import tpu_sc as plsc

VOCAB = 1000000
EMB = 64
B = 16384
L = 50
HID = 128

NC = 2
NS = 16
NW = NC * NS
BPW = B // NW
GCH = 128
NK = BPW // GCH

VBLK = 4096
NGRID = (VOCAB + VBLK - 1) // VBLK
VPAD = NGRID * VBLK
HSH = VBLK.bit_length() - 2


def _sc_bag_sums(x_flat, emb_table):
    mesh = plsc.VectorSubcoreMesh(core_axis_name="c", subcore_axis_name="s")

    @functools.partial(
        pl.kernel,
        out_type=jax.ShapeDtypeStruct((B, EMB), jnp.float32),
        mesh=mesh,
        name="bag_sums",
        compiler_params=pltpu.CompilerParams(
            use_tc_tiling_on_sc=False, needs_layout_passes=False
        ),
        scratch_types=[
            pltpu.VMEM((BPW * L,), jnp.int32),
            pltpu.VMEM((L, NK, GCH), jnp.int32),
            pltpu.VMEM((BPW, EMB), jnp.float32),
            pltpu.SemaphoreType.DMA,
        ],
    )
    def kern(x_hbm, tab_hbm, out_hbm, raw_v, idx_v, acc_v, sem):
        wid = lax.axis_index("s") * NC + lax.axis_index("c")
        pltpu.sync_copy(x_hbm.at[pl.ds(wid * (BPW * L), BPW * L)], raw_v)

        lane = lax.iota(jnp.int32, 16) * L

        def transpose_row(j):
            for kk in range(NK):
                for g in range(GCH // 16):
                    v = plsc.load_gather(
                        raw_v, [lane + (kk * GCH + g * 16) * L + j]
                    )
                    pv = (
                        (v & jnp.int32(~(VBLK - 1)))
                        + ((v & jnp.int32(VBLK // 2 - 1)) << 1)
                        + ((v >> HSH) & jnp.int32(1))
                    )
                    idx_v[j, kk, pl.ds(g * 16, 16)] = pv

        transpose_row(0)
        cps = [
            pltpu.async_copy(
                tab_hbm.at[idx_v.at[0, kk]],
                acc_v.at[pl.ds(kk * GCH, GCH)],
                sem,
            )
            for kk in range(NK)
        ]
        for cp in cps:
            cp.wait()

        @pl.loop(1, L)
        def _(j):
            transpose_row(j)
            for kk in range(NK):
                pltpu.async_copy(
                    tab_hbm.at[idx_v.at[j, kk]],
                    acc_v.at[pl.ds(kk * GCH, GCH)],
                    sem,
                    add=True,
                )

        @pl.loop(1, L)
        def _(j):
            for kk in range(NK):
                pltpu.make_async_copy(
                    tab_hbm.at[idx_v.at[0, kk]],
                    acc_v.at[pl.ds(kk * GCH, GCH)],
                    sem,
                ).wait()

        pltpu.sync_copy(acc_v, out_hbm.at[pl.ds(wid * BPW, BPW)])

    return kern(x_flat, emb_table)


def _tc_detile(tabT):
    def body(t_ref, o_ref):
        t = t_ref[...].T
        y = jnp.concatenate([t[: VBLK // 2, :], t[VBLK // 2 :, :]], axis=1)
        o_ref[...] = y.reshape(VBLK * EMB)

    return pl.pallas_call(
        body,
        grid=(NGRID,),
        in_specs=[pl.BlockSpec((EMB, VBLK), lambda i: (0, i))],
        out_specs=pl.BlockSpec((VBLK * EMB,), lambda i: (i,)),
        out_shape=jax.ShapeDtypeStruct((VPAD * EMB,), jnp.float32),
    )(tabT)


def _tc_mlp(sums, W1, b1, W2, b2):
    BLK = 2048

    def body(s_ref, w1_ref, b1_ref, w2_ref, b2_ref, o_ref):
        e = s_ref[...] * (1.0 / L)
        h = jnp.dot(e, w1_ref[...], preferred_element_type=jnp.float32)
        h = jnp.maximum(h + b1_ref[...], 0.0)
        p = jnp.dot(h, w2_ref[...], preferred_element_type=jnp.float32)
        o_ref[...] = jax.nn.sigmoid(p + b2_ref[...])

    return pl.pallas_call(
        body,
        grid=(B // BLK,),
        in_specs=[
            pl.BlockSpec((BLK, EMB), lambda i: (i, 0)),
            pl.BlockSpec((EMB, HID), lambda i: (0, 0)),
            pl.BlockSpec((1, HID), lambda i: (0, 0)),
            pl.BlockSpec((HID, 1), lambda i: (0, 0)),
            pl.BlockSpec((1, 1), lambda i: (0, 0)),
        ],
        out_specs=pl.BlockSpec((BLK, 1), lambda i: (i, 0)),
        out_shape=jax.ShapeDtypeStruct((B, 1), jnp.float32),
    )(sums, W1, b1, W2, b2)


def kernel(x, emb_table, W1, b1, W2, b2):
    tab_rm = _tc_detile(emb_table.T).reshape(VPAD, EMB)
    sums = _sc_bag_sums(x.reshape(B * L), tab_rm)
    return _tc_mlp(sums, W1, b1.reshape(1, HID), W2, b2.reshape(1, 1))


# speedup vs baseline: 2.1140x; 1.1750x over previous
import functools

import jax
import jax.numpy as jnp
from jax import lax
from jax.experimental import pallas as pl
from jax.experimental.pallas import tpu as pltpu
from jax.experimental.pallas import tpu_sc as plsc

VOCAB = 1000000
EMB = 64
B = 16384
L = 50
HID = 128

NC = 2
NS = 16
NW = NC * NS
BPW = B // NW
GCH = 128
NK = BPW // GCH

VBLK = 8192
NGRID = (VOCAB + VBLK - 1) // VBLK
VPAD = NGRID * VBLK
HSH = VBLK.bit_length() - 2


def _sc_bag_sums(x_flat, emb_table):
    mesh = plsc.VectorSubcoreMesh(core_axis_name="c", subcore_axis_name="s")

    @functools.partial(
        pl.kernel,
        out_type=jax.ShapeDtypeStruct((B, EMB), jnp.float32),
        mesh=mesh,
        name="bag_sums",
        compiler_params=pltpu.CompilerParams(
            use_tc_tiling_on_sc=False, needs_layout_passes=False
        ),
        scratch_types=[
            pltpu.VMEM((BPW * L,), jnp.int32),
            pltpu.VMEM((L, NK, GCH), jnp.int32),
            pltpu.VMEM((BPW, EMB), jnp.float32),
            pltpu.SemaphoreType.DMA,
        ],
    )
    def kern(x_hbm, tab_hbm, out_hbm, raw_v, idx_v, acc_v, sem):
        wid = lax.axis_index("s") * NC + lax.axis_index("c")
        pltpu.sync_copy(x_hbm.at[pl.ds(wid * (BPW * L), BPW * L)], raw_v)

        lane = lax.iota(jnp.int32, 16) * L

        def transpose_row(j):
            for kk in range(NK):
                for g in range(GCH // 16):
                    v = plsc.load_gather(
                        raw_v, [lane + (kk * GCH + g * 16) * L + j]
                    )
                    pv = (
                        (v & jnp.int32(~(VBLK - 1)))
                        + ((v & jnp.int32(VBLK // 2 - 1)) << 1)
                        + ((v >> HSH) & jnp.int32(1))
                    )
                    idx_v[j, kk, pl.ds(g * 16, 16)] = pv

        transpose_row(0)
        cps = [
            pltpu.async_copy(
                tab_hbm.at[idx_v.at[0, kk]],
                acc_v.at[pl.ds(kk * GCH, GCH)],
                sem,
            )
            for kk in range(NK)
        ]
        for cp in cps:
            cp.wait()

        @pl.loop(1, L)
        def _(j):
            transpose_row(j)
            for kk in range(NK):
                pltpu.async_copy(
                    tab_hbm.at[idx_v.at[j, kk]],
                    acc_v.at[pl.ds(kk * GCH, GCH)],
                    sem,
                    add=True,
                )

        @pl.loop(1, L)
        def _(j):
            for kk in range(NK):
                pltpu.make_async_copy(
                    tab_hbm.at[idx_v.at[0, kk]],
                    acc_v.at[pl.ds(kk * GCH, GCH)],
                    sem,
                ).wait()

        pltpu.sync_copy(acc_v, out_hbm.at[pl.ds(wid * BPW, BPW)])

    return kern(x_flat, emb_table)


def _tc_detile(tabT):
    def body(t_ref, o_ref):
        t = t_ref[...].T
        y = jnp.concatenate([t[: VBLK // 2, :], t[VBLK // 2 :, :]], axis=1)
        o_ref[...] = y.reshape(VBLK * EMB)

    return pl.pallas_call(
        body,
        grid=(NGRID,),
        in_specs=[pl.BlockSpec((EMB, VBLK), lambda i: (0, i))],
        out_specs=pl.BlockSpec((VBLK * EMB,), lambda i: (i,)),
        out_shape=jax.ShapeDtypeStruct((VPAD * EMB,), jnp.float32),
    )(tabT)


def _tc_mlp(sums, W1, b1, W2, b2):
    BLK = 2048

    def body(s_ref, w1_ref, b1_ref, w2_ref, b2_ref, o_ref):
        e = s_ref[...] * (1.0 / L)
        h = jnp.dot(e, w1_ref[...], preferred_element_type=jnp.float32)
        h = jnp.maximum(h + b1_ref[...], 0.0)
        p = jnp.dot(h, w2_ref[...], preferred_element_type=jnp.float32)
        o_ref[...] = jax.nn.sigmoid(p + b2_ref[...])

    return pl.pallas_call(
        body,
        grid=(B // BLK,),
        in_specs=[
            pl.BlockSpec((BLK, EMB), lambda i: (i, 0)),
            pl.BlockSpec((EMB, HID), lambda i: (0, 0)),
            pl.BlockSpec((1, HID), lambda i: (0, 0)),
            pl.BlockSpec((HID, 1), lambda i: (0, 0)),
            pl.BlockSpec((1, 1), lambda i: (0, 0)),
        ],
        out_specs=pl.BlockSpec((BLK, 1), lambda i: (i, 0)),
        out_shape=jax.ShapeDtypeStruct((B, 1), jnp.float32),
    )(sums, W1, b1, W2, b2)


def kernel(x, emb_table, W1, b1, W2, b2):
    tab_rm = _tc_detile(emb_table.T).reshape(VPAD, EMB)
    sums = _sc_bag_sums(x.reshape(B * L), tab_rm)
    return _tc_mlp(sums, W1, b1.reshape(1, HID), W2, b2.reshape(1, 1))


# speedup vs baseline: 2.2975x; 1.0868x over previous
import functools

import jax
import jax.numpy as jnp
from jax import lax
from jax.experimental import pallas as pl
from jax.experimental.pallas import tpu as pltpu
from jax.experimental.pallas import tpu_sc as plsc

VOCAB = 1000000
EMB = 64
B = 16384
L = 50
HID = 128

NC = 2
NS = 16
NW = NC * NS
BPW = B // NW
GCH = 128
NK = BPW // GCH

VBLK = 16384
NGRID = (VOCAB + VBLK - 1) // VBLK
VPAD = NGRID * VBLK
HSH = VBLK.bit_length() - 2


def _sc_bag_sums(x_flat, emb_table):
    mesh = plsc.VectorSubcoreMesh(core_axis_name="c", subcore_axis_name="s")

    @functools.partial(
        pl.kernel,
        out_type=jax.ShapeDtypeStruct((B, EMB), jnp.float32),
        mesh=mesh,
        name="bag_sums",
        compiler_params=pltpu.CompilerParams(
            use_tc_tiling_on_sc=False, needs_layout_passes=False
        ),
        scratch_types=[
            pltpu.VMEM((BPW * L,), jnp.int32),
            pltpu.VMEM((L, NK, GCH), jnp.int32),
            pltpu.VMEM((BPW, EMB), jnp.float32),
            pltpu.SemaphoreType.DMA,
        ],
    )
    def kern(x_hbm, tab_hbm, out_hbm, raw_v, idx_v, acc_v, sem):
        wid = lax.axis_index("s") * NC + lax.axis_index("c")
        pltpu.sync_copy(x_hbm.at[pl.ds(wid * (BPW * L), BPW * L)], raw_v)

        lane = lax.iota(jnp.int32, 16) * L

        def transpose_row(j):
            for kk in range(NK):
                for g in range(GCH // 16):
                    v = plsc.load_gather(
                        raw_v, [lane + (kk * GCH + g * 16) * L + j]
                    )
                    pv = (
                        (v & jnp.int32(~(VBLK - 1)))
                        + ((v & jnp.int32(VBLK // 2 - 1)) << 1)
                        + ((v >> HSH) & jnp.int32(1))
                    )
                    idx_v[j, kk, pl.ds(g * 16, 16)] = pv

        transpose_row(0)
        cps = [
            pltpu.async_copy(
                tab_hbm.at[idx_v.at[0, kk]],
                acc_v.at[pl.ds(kk * GCH, GCH)],
                sem,
            )
            for kk in range(NK)
        ]
        for cp in cps:
            cp.wait()

        @pl.loop(1, L)
        def _(j):
            transpose_row(j)
            for kk in range(NK):
                pltpu.async_copy(
                    tab_hbm.at[idx_v.at[j, kk]],
                    acc_v.at[pl.ds(kk * GCH, GCH)],
                    sem,
                    add=True,
                )

        @pl.loop(1, L)
        def _(j):
            for kk in range(NK):
                pltpu.make_async_copy(
                    tab_hbm.at[idx_v.at[0, kk]],
                    acc_v.at[pl.ds(kk * GCH, GCH)],
                    sem,
                ).wait()

        pltpu.sync_copy(acc_v, out_hbm.at[pl.ds(wid * BPW, BPW)])

    return kern(x_flat, emb_table)


def _tc_detile(tabT):
    def body(t_ref, o_ref):
        t = t_ref[...].T
        y = jnp.concatenate([t[: VBLK // 2, :], t[VBLK // 2 :, :]], axis=1)
        o_ref[...] = y.reshape(VBLK * EMB)

    return pl.pallas_call(
        body,
        grid=(NGRID,),
        in_specs=[pl.BlockSpec((EMB, VBLK), lambda i: (0, i))],
        out_specs=pl.BlockSpec((VBLK * EMB,), lambda i: (i,)),
        out_shape=jax.ShapeDtypeStruct((VPAD * EMB,), jnp.float32),
    )(tabT)


def _tc_mlp(sums, W1, b1, W2, b2):
    BLK = 2048

    def body(s_ref, w1_ref, b1_ref, w2_ref, b2_ref, o_ref):
        e = s_ref[...] * (1.0 / L)
        h = jnp.dot(e, w1_ref[...], preferred_element_type=jnp.float32)
        h = jnp.maximum(h + b1_ref[...], 0.0)
        p = jnp.dot(h, w2_ref[...], preferred_element_type=jnp.float32)
        o_ref[...] = jax.nn.sigmoid(p + b2_ref[...])

    return pl.pallas_call(
        body,
        grid=(B // BLK,),
        in_specs=[
            pl.BlockSpec((BLK, EMB), lambda i: (i, 0)),
            pl.BlockSpec((EMB, HID), lambda i: (0, 0)),
            pl.BlockSpec((1, HID), lambda i: (0, 0)),
            pl.BlockSpec((HID, 1), lambda i: (0, 0)),
            pl.BlockSpec((1, 1), lambda i: (0, 0)),
        ],
        out_specs=pl.BlockSpec((BLK, 1), lambda i: (i, 0)),
        out_shape=jax.ShapeDtypeStruct((B, 1), jnp.float32),
    )(sums, W1, b1, W2, b2)


def kernel(x, emb_table, W1, b1, W2, b2):
    tab_rm = _tc_detile(emb_table.T).reshape(VPAD, EMB)
    sums = _sc_bag_sums(x.reshape(B * L), tab_rm)
    return _tc_mlp(sums, W1, b1.reshape(1, HID), W2, b2.reshape(1, 1))


# speedup vs baseline: 2.3968x; 1.0432x over previous
import functools

import jax
import jax.numpy as jnp
from jax import lax
from jax.experimental import pallas as pl
from jax.experimental.pallas import tpu as pltpu
from jax.experimental.pallas import tpu_sc as plsc

VOCAB = 1000000
EMB = 64
B = 16384
L = 50
HID = 128

NC = 2
NS = 16
NW = NC * NS
BPW = B // NW
GCH = 128
NK = BPW // GCH

VBLK = 32768
NGRID = (VOCAB + VBLK - 1) // VBLK
VPAD = NGRID * VBLK
HSH = VBLK.bit_length() - 2


def _sc_bag_sums(x_flat, emb_table):
    mesh = plsc.VectorSubcoreMesh(core_axis_name="c", subcore_axis_name="s")

    @functools.partial(
        pl.kernel,
        out_type=jax.ShapeDtypeStruct((B, EMB), jnp.float32),
        mesh=mesh,
        name="bag_sums",
        compiler_params=pltpu.CompilerParams(
            use_tc_tiling_on_sc=False, needs_layout_passes=False
        ),
        scratch_types=[
            pltpu.VMEM((BPW * L,), jnp.int32),
            pltpu.VMEM((L, NK, GCH), jnp.int32),
            pltpu.VMEM((BPW, EMB), jnp.float32),
            pltpu.SemaphoreType.DMA,
        ],
    )
    def kern(x_hbm, tab_hbm, out_hbm, raw_v, idx_v, acc_v, sem):
        wid = lax.axis_index("s") * NC + lax.axis_index("c")
        pltpu.sync_copy(x_hbm.at[pl.ds(wid * (BPW * L), BPW * L)], raw_v)

        lane = lax.iota(jnp.int32, 16) * L

        def transpose_row(j):
            for kk in range(NK):
                for g in range(GCH // 16):
                    v = plsc.load_gather(
                        raw_v, [lane + (kk * GCH + g * 16) * L + j]
                    )
                    pv = (
                        (v & jnp.int32(~(VBLK - 1)))
                        + ((v & jnp.int32(VBLK // 2 - 1)) << 1)
                        + ((v >> HSH) & jnp.int32(1))
                    )
                    idx_v[j, kk, pl.ds(g * 16, 16)] = pv

        transpose_row(0)
        cps = [
            pltpu.async_copy(
                tab_hbm.at[idx_v.at[0, kk]],
                acc_v.at[pl.ds(kk * GCH, GCH)],
                sem,
            )
            for kk in range(NK)
        ]
        for cp in cps:
            cp.wait()

        @pl.loop(1, L)
        def _(j):
            transpose_row(j)
            for kk in range(NK):
                pltpu.async_copy(
                    tab_hbm.at[idx_v.at[j, kk]],
                    acc_v.at[pl.ds(kk * GCH, GCH)],
                    sem,
                    add=True,
                )

        @pl.loop(1, L)
        def _(j):
            for kk in range(NK):
                pltpu.make_async_copy(
                    tab_hbm.at[idx_v.at[0, kk]],
                    acc_v.at[pl.ds(kk * GCH, GCH)],
                    sem,
                ).wait()

        pltpu.sync_copy(acc_v, out_hbm.at[pl.ds(wid * BPW, BPW)])

    return kern(x_flat, emb_table)


def _tc_detile(tabT):
    def body(t_ref, o_ref):
        t = t_ref[...].T
        y = jnp.concatenate([t[: VBLK // 2, :], t[VBLK // 2 :, :]], axis=1)
        o_ref[...] = y.reshape(VBLK * EMB)

    return pl.pallas_call(
        body,
        grid=(NGRID,),
        in_specs=[pl.BlockSpec((EMB, VBLK), lambda i: (0, i))],
        out_specs=pl.BlockSpec((VBLK * EMB,), lambda i: (i,)),
        out_shape=jax.ShapeDtypeStruct((VPAD * EMB,), jnp.float32),
    )(tabT)


def _tc_mlp(sums, W1, b1, W2, b2):
    BLK = 2048

    def body(s_ref, w1_ref, b1_ref, w2_ref, b2_ref, o_ref):
        e = s_ref[...] * (1.0 / L)
        h = jnp.dot(e, w1_ref[...], preferred_element_type=jnp.float32)
        h = jnp.maximum(h + b1_ref[...], 0.0)
        p = jnp.dot(h, w2_ref[...], preferred_element_type=jnp.float32)
        o_ref[...] = jax.nn.sigmoid(p + b2_ref[...])

    return pl.pallas_call(
        body,
        grid=(B // BLK,),
        in_specs=[
            pl.BlockSpec((BLK, EMB), lambda i: (i, 0)),
            pl.BlockSpec((EMB, HID), lambda i: (0, 0)),
            pl.BlockSpec((1, HID), lambda i: (0, 0)),
            pl.BlockSpec((HID, 1), lambda i: (0, 0)),
            pl.BlockSpec((1, 1), lambda i: (0, 0)),
        ],
        out_specs=pl.BlockSpec((BLK, 1), lambda i: (i, 0)),
        out_shape=jax.ShapeDtypeStruct((B, 1), jnp.float32),
    )(sums, W1, b1, W2, b2)


def kernel(x, emb_table, W1, b1, W2, b2):
    tab_rm = _tc_detile(emb_table.T).reshape(VPAD, EMB)
    sums = _sc_bag_sums(x.reshape(B * L), tab_rm)
    return _tc_mlp(sums, W1, b1.reshape(1, HID), W2, b2.reshape(1, 1))
